# Initial kernel scaffold; baseline (speedup 1.0000x reference)
import jax
import jax.numpy as jnp
from jax.experimental import pallas as pl


def kernel(feature, edge_index, W, b):
    raise NotImplementedError("write your pallas kernel here")


# speedup vs baseline: 7.5115x; 7.5115x over previous
import functools

import jax
import jax.numpy as jnp
from jax import lax
from jax.experimental import pallas as pl
from jax.experimental.pallas import tpu as pltpu
from jax.experimental.pallas import tpu_sc as plsc

N_NODES = 10000
N_EDGES = 320000
D_FEAT = 128
OUT_FEATS = 128
FAT = 144

NC = 2
NS = 16
NW = NC * NS
EDGES_PER_W = N_EDGES // NW
BLK = 125
NBLK = EDGES_PER_W // BLK
ROWS_PER_TILE = N_NODES // NS


def _sc_segment_sum(fat_feature, src_r, dst_r, zeros_hbm):
    mesh = plsc.VectorSubcoreMesh(core_axis_name="c", subcore_axis_name="s")

    @functools.partial(
        pl.kernel,
        mesh=mesh,
        compiler_params=pltpu.CompilerParams(use_tc_tiling_on_sc=False),
        out_type=jax.ShapeDtypeStruct((NC, N_NODES, FAT), jnp.float32),
        scratch_types=[
            pltpu.VMEM((NBLK, BLK), jnp.int32),
            pltpu.VMEM((NBLK, BLK), jnp.int32),
            pltpu.VMEM((BLK, FAT), jnp.float32),
            pltpu.VMEM_SHARED((N_NODES, FAT), jnp.float32),
            pltpu.SemaphoreType.DMA,
        ],
    )
    def kern(fat_hbm, src_hbm, dst_hbm, zero_hbm, out_hbm,
             src_v, dst_v, rows_v, acc_sh, sem):
        c = lax.axis_index("c")
        s = lax.axis_index("s")
        wid = s * NC + c

        pltpu.sync_copy(src_hbm.at[wid], src_v)
        pltpu.sync_copy(dst_hbm.at[wid], dst_v)

        pltpu.sync_copy(zero_hbm, acc_sh.at[pl.ds(s * ROWS_PER_TILE, ROWS_PER_TILE)])
        plsc.subcore_barrier()

        def blk(j, carry):
            pltpu.async_copy(fat_hbm.at[src_v.at[j]], rows_v, sem).wait()
            pltpu.sync_copy(rows_v, acc_sh.at[dst_v.at[j]], add=True)
            return carry

        lax.fori_loop(0, NBLK, blk, 0)
        plsc.subcore_barrier()

        pltpu.sync_copy(
            acc_sh.at[pl.ds(s * ROWS_PER_TILE, ROWS_PER_TILE)],
            out_hbm.at[c].at[pl.ds(s * ROWS_PER_TILE, ROWS_PER_TILE)],
        )

    return kern(fat_feature, src_r, dst_r, zeros_hbm)


def _tc_mean_linear(partials, feature, w1t, w2t, b2d):
    blk_rows = 1000
    grid = (N_NODES // blk_rows,)

    def body(p_ref, f_ref, w1_ref, w2_ref, b_ref, o_ref):
        p0 = p_ref[0]
        p1 = p_ref[1]
        summed = p0[:, :D_FEAT] + p1[:, :D_FEAT]
        cnt = p0[:, D_FEAT:D_FEAT + 1] + p1[:, D_FEAT:D_FEAT + 1]
        h = summed / jnp.maximum(cnt, 1.0)
        o_ref[...] = (
            jnp.dot(h, w1_ref[...], preferred_element_type=jnp.float32)
            + jnp.dot(f_ref[...], w2_ref[...], preferred_element_type=jnp.float32)
            + b_ref[...]
        )

    return pl.pallas_call(
        body,
        grid=grid,
        in_specs=[
            pl.BlockSpec((NC, blk_rows, FAT), lambda i: (0, i, 0)),
            pl.BlockSpec((blk_rows, D_FEAT), lambda i: (i, 0)),
            pl.BlockSpec((D_FEAT, OUT_FEATS), lambda i: (0, 0)),
            pl.BlockSpec((D_FEAT, OUT_FEATS), lambda i: (0, 0)),
            pl.BlockSpec((1, OUT_FEATS), lambda i: (0, 0)),
        ],
        out_specs=pl.BlockSpec((blk_rows, OUT_FEATS), lambda i: (i, 0)),
        out_shape=jax.ShapeDtypeStruct((N_NODES, OUT_FEATS), jnp.float32),
    )(partials, feature, w1t, w2t, b2d)


def kernel(feature, edge_index, W, b):
    ei = edge_index.astype(jnp.int32)
    src_r = ei[0].reshape(NW, NBLK, BLK)
    dst_r = ei[1].reshape(NW, NBLK, BLK)

    fat = jnp.zeros((N_NODES, FAT), jnp.float32)
    fat = fat.at[:, :D_FEAT].set(feature)
    fat = fat.at[:, D_FEAT].set(1.0)

    zeros_hbm = jnp.zeros((ROWS_PER_TILE, FAT), jnp.float32)

    partials = _sc_segment_sum(fat, src_r, dst_r, zeros_hbm)

    w1t = W[:, :D_FEAT].T
    w2t = W[:, D_FEAT:].T
    b2d = b.reshape(1, OUT_FEATS)
    return _tc_mean_linear(partials, feature, w1t, w2t, b2d)
